# baseline (speedup 1.0000x reference)
import functools

import jax
import jax.numpy as jnp
from jax import lax
from jax.experimental import pallas as pl
from jax.experimental.pallas import tpu as pltpu
from jax.experimental.pallas import tpu_sc as plsc

_NC = 2
_NS = 16
_NW = _NC * _NS

_CHUNK = 512


def _build(B, D, b_per_w, ch):
    nch = b_per_w // ch
    mesh = plsc.VectorSubcoreMesh(
        core_axis_name="c", subcore_axis_name="s",
        num_cores=_NC, num_subcores=_NS)

    @functools.partial(
        pl.kernel,
        out_type=jax.ShapeDtypeStruct((B, D), jnp.float32),
        mesh=mesh,
        scratch_types=[
            pltpu.VMEM((b_per_w,), jnp.int32),
            pltpu.VMEM((ch, D), jnp.float32),
            pltpu.SemaphoreType.DMA,
        ],
        compiler_params=pltpu.CompilerParams(use_tc_tiling_on_sc=False),
    )
    def k(idx_hbm, table_hbm, out_hbm, idx_v, rows_v, sem):
        wid = lax.axis_index("s") * _NC + lax.axis_index("c")
        base = wid * b_per_w
        pltpu.sync_copy(idx_hbm.at[pl.ds(base, b_per_w)], idx_v)

        def chunk(j, carry):
            off = j * ch
            pltpu.async_copy(
                table_hbm.at[idx_v.at[pl.ds(off, ch)]], rows_v, sem).wait()
            pltpu.sync_copy(rows_v, out_hbm.at[pl.ds(base + off, ch)])
            return carry

        lax.fori_loop(0, nch, chunk, 0)

    return k


@jax.jit
def kernel(x, table):
    b0, b1 = x.shape
    B = b0 * b1
    D = table.shape[1]
    idx = x.reshape(B).astype(jnp.int32)
    out = _build(B, D, B // _NW, _CHUNK)(idx, table)
    return out.reshape(b0, b1, D)


# speedup vs baseline: 1.0226x; 1.0226x over previous
import functools

import jax
import jax.numpy as jnp
from jax import lax
from jax.experimental import pallas as pl
from jax.experimental.pallas import tpu as pltpu
from jax.experimental.pallas import tpu_sc as plsc

_NC = 2
_NS = 16
_NW = _NC * _NS

_CHUNK = 400
_NBUF = 4


def _build(B, D, b_per_w, ch):
    nch = b_per_w // ch
    assert nch % _NBUF == 0 and nch >= 2 * _NBUF
    mesh = plsc.VectorSubcoreMesh(
        core_axis_name="c", subcore_axis_name="s",
        num_cores=_NC, num_subcores=_NS)

    @functools.partial(
        pl.kernel,
        out_type=jax.ShapeDtypeStruct((B, D), jnp.float32),
        mesh=mesh,
        scratch_types=[
            pltpu.VMEM((b_per_w,), jnp.int32),
            [pltpu.VMEM((ch, D), jnp.float32)] * _NBUF,
            [pltpu.SemaphoreType.DMA] * _NBUF,
            [pltpu.SemaphoreType.DMA] * _NBUF,
        ],
        compiler_params=pltpu.CompilerParams(use_tc_tiling_on_sc=False),
    )
    def k(idx_hbm, table_hbm, out_hbm, idx_v, rows, gsem, wsem):
        wid = lax.axis_index("s") * _NC + lax.axis_index("c")
        base = wid * b_per_w
        pltpu.sync_copy(idx_hbm.at[pl.ds(base, b_per_w)], idx_v)

        def gather_start(g, b):
            pltpu.async_copy(
                table_hbm.at[idx_v.at[pl.ds(g * ch, ch)]], rows[b], gsem[b])

        def gather_wait(b):
            pltpu.make_async_copy(
                table_hbm.at[pl.ds(0, ch)], rows[b], gsem[b]).wait()

        def write_start(g, b):
            pltpu.async_copy(
                rows[b], out_hbm.at[pl.ds(base + g * ch, ch)], wsem[b])

        def write_wait(b):
            pltpu.make_async_copy(
                rows[b], out_hbm.at[pl.ds(base, ch)], wsem[b]).wait()

        gather_start(0, 0)
        gather_start(1, 1)

        def outer(j, carry):
            for b in range(_NBUF):
                g = j * _NBUF + b

                gather_wait(b)
                write_start(g, b)

                @pl.when(g >= 2)
                def _():
                    write_wait((b + _NBUF - 2) % _NBUF)

                @pl.when(g + 2 < nch)
                def _():
                    gather_start(g + 2, (b + 2) % _NBUF)

            return carry

        lax.fori_loop(0, nch // _NBUF, outer, 0)
        write_wait((nch - 2) % _NBUF)
        write_wait((nch - 1) % _NBUF)

    return k


@jax.jit
def kernel(x, table):
    b0, b1 = x.shape
    B = b0 * b1
    D = table.shape[1]
    idx = x.reshape(B).astype(jnp.int32)
    out = _build(B, D, B // _NW, _CHUNK)(idx, table)
    return out.reshape(b0, b1, D)


# speedup vs baseline: 1.2460x; 1.2185x over previous
import functools

import jax
import jax.numpy as jnp
from jax import lax
from jax.experimental import pallas as pl
from jax.experimental.pallas import tpu as pltpu
from jax.experimental.pallas import tpu_sc as plsc

_NC = 2
_NS = 16
_NW = _NC * _NS

_CHUNK = 128
_NBUF = 4


def _build(B, D, b_per_w, ch):
    nch = b_per_w // ch
    assert nch % _NBUF == 0 and nch >= 2 * _NBUF
    mesh = plsc.VectorSubcoreMesh(
        core_axis_name="c", subcore_axis_name="s",
        num_cores=_NC, num_subcores=_NS)

    @functools.partial(
        pl.kernel,
        out_type=jax.ShapeDtypeStruct((B, D), jnp.float32),
        mesh=mesh,
        scratch_types=[
            pltpu.VMEM((b_per_w,), jnp.int32),
            [pltpu.VMEM((ch, D), jnp.float32)] * _NBUF,
            [pltpu.SemaphoreType.DMA] * _NBUF,
            [pltpu.SemaphoreType.DMA] * _NBUF,
        ],
        compiler_params=pltpu.CompilerParams(use_tc_tiling_on_sc=False),
    )
    def k(idx_hbm, table_hbm, out_hbm, idx_v, rows, gsem, wsem):
        wid = lax.axis_index("s") * _NC + lax.axis_index("c")
        base = wid * b_per_w
        pltpu.sync_copy(idx_hbm.at[pl.ds(base, b_per_w)], idx_v)

        def gather_start(g, b):
            pltpu.async_copy(
                table_hbm.at[idx_v.at[pl.ds(g * ch, ch)]], rows[b], gsem[b])

        def gather_wait(b):
            pltpu.make_async_copy(
                table_hbm.at[pl.ds(0, ch)], rows[b], gsem[b]).wait()

        def write_start(g, b):
            pltpu.async_copy(
                rows[b], out_hbm.at[pl.ds(base + g * ch, ch)], wsem[b])

        def write_wait(b):
            pltpu.make_async_copy(
                rows[b], out_hbm.at[pl.ds(base, ch)], wsem[b]).wait()

        gather_start(0, 0)
        gather_start(1, 1)

        def outer(j, carry):
            for b in range(_NBUF):
                g = j * _NBUF + b

                gather_wait(b)
                write_start(g, b)

                @pl.when(g >= 2)
                def _():
                    write_wait((b + _NBUF - 2) % _NBUF)

                @pl.when(g + 2 < nch)
                def _():
                    gather_start(g + 2, (b + 2) % _NBUF)

            return carry

        lax.fori_loop(0, nch // _NBUF, outer, 0)
        write_wait((nch - 2) % _NBUF)
        write_wait((nch - 1) % _NBUF)

    return k


@jax.jit
def kernel(x, table):
    b0, b1 = x.shape
    B = b0 * b1
    d = table.shape[1]
    dp = 2 * d
    idx = x.reshape(B).astype(jnp.int32)
    table_p = jnp.pad(table, ((0, 0), (0, dp - d)))
    out = _build(B, dp, B // _NW, _CHUNK)(idx, table_p)
    return out.reshape(b0, b1, dp)[:, :, :d]
